# baseline (speedup 1.0000x reference)
import functools

import jax
import jax.numpy as jnp
from jax import lax
from jax.experimental import pallas as pl
from jax.experimental.pallas import tpu as pltpu
from jax.experimental.pallas import tpu_sc as plsc


def _detile_kernel(V, E, NC, NS):
    NW = NC * NS
    n_full = V // 128
    rem = V - n_full * 128
    eh_tiles = E // 8
    mesh = plsc.VectorSubcoreMesh(core_axis_name="c", subcore_axis_name="s")

    G = 4
    W = G * 128
    cpw = (n_full // NW) & ~(G - 1)
    last_cnt = n_full - (NW - 1) * cpw

    @functools.partial(
        pl.kernel,
        mesh=mesh,
        compiler_params=pltpu.CompilerParams(needs_layout_passes=False),
        out_type=jax.ShapeDtypeStruct((V * E,), jnp.float32),
        scratch_types=[
            pltpu.VMEM((E, W + 1), jnp.float32),
            pltpu.VMEM((E, W + 1), jnp.float32),
            pltpu.VMEM((W * E,), jnp.float32),
            pltpu.VMEM((max(rem, 1) * E,), jnp.float32),
            pltpu.SemaphoreType.DMA,
            pltpu.SemaphoreType.DMA,
        ],
    )
    def k(tokT_hbm, tail_hbm, out_hbm, tinA, tinB, tout, tail_v, semA, semB):
        wid = lax.axis_index("s") * NC + lax.axis_index("c")
        lane = jax.lax.broadcasted_iota(jnp.int32, (16,), 0)
        patterns = [lane + 16 * h for h in range(E // 16)]

        start = wid * cpw
        n_g = jnp.where(wid == NW - 1, last_cnt // G, cpw // G)

        def fire(g, tin, sem):
            c0 = (start + g * G) * 128
            for eh in range(eh_tiles):
                pltpu.async_copy(
                    tokT_hbm.at[pl.ds(8 * eh, 8), pl.ds(c0, W)],
                    tin.at[pl.ds(8 * eh, 8), pl.ds(0, W)],
                    sem,
                )

        def drain(g, tin, sem):
            c0 = (start + g * G) * 128
            for eh in range(eh_tiles):
                pltpu.make_async_copy(
                    tokT_hbm.at[pl.ds(8 * eh, 8), pl.ds(c0, W)],
                    tin.at[pl.ds(8 * eh, 8), pl.ds(0, W)],
                    sem,
                ).wait()

        def work(g, tin, sem):
            @pl.when(g + 1 < n_g)
            def _():
                other = tinB if tin is tinA else tinA
                osem = semB if sem is semA else semA
                fire(g + 1, other, osem)

            drain(g, tin, sem)

            UNR = 8

            def col_body(j8, jv):
                for u in range(UNR):
                    for h in range(E // 16):
                        v = plsc.load_gather(tin, [patterns[h], jv + u])
                        tout[pl.ds((j8 * UNR + u) * E + 16 * h, 16)] = v
                return jv + UNR

            lax.fori_loop(0, W // UNR, col_body, lane * 0)
            pltpu.sync_copy(
                tout, out_hbm.at[pl.ds((start + g * G) * 128 * E, W * E)]
            )

        @pl.when(n_g > 0)
        def _():
            fire(0, tinA, semA)

        def g_body(g, carry):
            @pl.when((g % 2 == 0) & (g < n_g))
            def _():
                work(g, tinA, semA)

            @pl.when((g % 2 == 1) & (g < n_g))
            def _():
                work(g, tinB, semB)

            return carry

        lax.fori_loop(0, last_cnt // G, g_body, 0)

        if rem:
            @pl.when(wid == NW - 1)
            def _():
                pltpu.sync_copy(tail_hbm, tail_v)
                pltpu.sync_copy(
                    tail_v, out_hbm.at[pl.ds(n_full * 128 * E, rem * E)]
                )

    return k


def _emb_kernel(N, E, L, NC, NS, CH):
    NW = NC * NS
    rows_per_w = N // NW
    n_ch = rows_per_w // CH
    reps = CH // L
    mesh = plsc.VectorSubcoreMesh(core_axis_name="c", subcore_axis_name="s")

    @functools.partial(
        pl.kernel,
        mesh=mesh,
        compiler_params=pltpu.CompilerParams(use_tc_tiling_on_sc=False),
        out_type=jax.ShapeDtypeStruct((N, 128), jnp.float32),
        scratch_types=[
            pltpu.VMEM((CH,), jnp.int32),
            pltpu.VMEM((CH, E), jnp.float32),
            pltpu.VMEM((L, E), jnp.float32),
            pltpu.SemaphoreType.DMA,
        ],
    )
    def k(x_hbm, tok_hbm, pos_hbm, out_hbm, idx_v, rows_v, pos_v, sem):
        wid = lax.axis_index("s") * NC + lax.axis_index("c")
        base = wid * rows_per_w
        pltpu.sync_copy(pos_hbm, pos_v)

        def chunk_body(c, carry):
            cb = base + c * CH
            pltpu.sync_copy(x_hbm.at[pl.ds(cb, CH)], idx_v)
            pltpu.async_copy(tok_hbm.at[idx_v], rows_v, sem).wait()

            def pos_body(p, carry2):
                for h in range(E // 16):
                    cs = pl.ds(h * 16, 16)
                    pv = pos_v[p, cs]
                    for j in range(reps):
                        r = j * L + p
                        rows_v[r, cs] = rows_v[r, cs] + pv
                return carry2

            lax.fori_loop(0, L, pos_body, 0, unroll=2)
            pltpu.sync_copy(rows_v, out_hbm.at[pl.ds(cb, CH), pl.ds(0, E)])
            return carry

        lax.fori_loop(0, n_ch, chunk_body, 0)

    return k


def kernel(x, token_table, pos_table):
    B, L = x.shape
    V, E = token_table.shape
    N = B * L
    x_flat = x.reshape(N).astype(jnp.int32)
    detile = _detile_kernel(V, E, NC=2, NS=16)
    n_full = V // 128
    tail = token_table[n_full * 128:].reshape(-1)
    tok_flat = detile(token_table.T, tail)
    k = _emb_kernel(N, E, L, NC=2, NS=16, CH=8 * L)
    out = k(x_flat, tok_flat.reshape(V, E), pos_table)
    return out[:, :E].reshape(B, L, E)


# speedup vs baseline: 1.0003x; 1.0003x over previous
import functools

import jax
import jax.numpy as jnp
from jax import lax
from jax.experimental import pallas as pl
from jax.experimental.pallas import tpu as pltpu
from jax.experimental.pallas import tpu_sc as plsc


def _detile_kernel(V, E, NC, NS):
    NW = NC * NS
    n_full = V // 128
    rem = V - n_full * 128
    eh_tiles = E // 8
    mesh = plsc.VectorSubcoreMesh(core_axis_name="c", subcore_axis_name="s")

    G = 4
    W = G * 128
    cpw = (n_full // NW) & ~(G - 1)
    last_cnt = n_full - (NW - 1) * cpw

    @functools.partial(
        pl.kernel,
        mesh=mesh,
        compiler_params=pltpu.CompilerParams(needs_layout_passes=False),
        out_type=jax.ShapeDtypeStruct((V * E,), jnp.float32),
        scratch_types=[
            pltpu.VMEM((E, W + 16), jnp.float32),
            pltpu.VMEM((E, W + 16), jnp.float32),
            pltpu.VMEM((W * E,), jnp.float32),
            pltpu.VMEM((max(rem, 1) * E,), jnp.float32),
            pltpu.SemaphoreType.DMA,
            pltpu.SemaphoreType.DMA,
        ],
    )
    def k(tokT_hbm, tail_hbm, out_hbm, tinA, tinB, tout, tail_v, semA, semB):
        wid = lax.axis_index("s") * NC + lax.axis_index("c")
        lane = jax.lax.broadcasted_iota(jnp.int32, (16,), 0)
        patterns = [lane + 16 * h for h in range(E // 16)]

        start = wid * cpw
        n_g = jnp.where(wid == NW - 1, last_cnt // G, cpw // G)

        def fire(g, tin, sem):
            c0 = (start + g * G) * 128
            for eh in range(eh_tiles):
                pltpu.async_copy(
                    tokT_hbm.at[pl.ds(8 * eh, 8), pl.ds(c0, W)],
                    tin.at[pl.ds(8 * eh, 8), pl.ds(0, W)],
                    sem,
                )

        def drain(g, tin, sem):
            c0 = (start + g * G) * 128
            for eh in range(eh_tiles):
                pltpu.make_async_copy(
                    tokT_hbm.at[pl.ds(8 * eh, 8), pl.ds(c0, W)],
                    tin.at[pl.ds(8 * eh, 8), pl.ds(0, W)],
                    sem,
                ).wait()

        def work(g, tin, sem):
            @pl.when(g + 1 < n_g)
            def _():
                other = tinB if tin is tinA else tinA
                osem = semB if sem is semA else semA
                fire(g + 1, other, osem)

            drain(g, tin, sem)

            UNR = 8

            def col_body(j8, jv):
                for u in range(UNR):
                    for h in range(E // 16):
                        v = plsc.load_gather(tin, [patterns[h], jv + u])
                        tout[pl.ds((j8 * UNR + u) * E + 16 * h, 16)] = v
                return jv + UNR

            lax.fori_loop(0, W // UNR, col_body, lane * 0)
            pltpu.sync_copy(
                tout, out_hbm.at[pl.ds((start + g * G) * 128 * E, W * E)]
            )

        @pl.when(n_g > 0)
        def _():
            fire(0, tinA, semA)

        def g_body(g, carry):
            @pl.when((g % 2 == 0) & (g < n_g))
            def _():
                work(g, tinA, semA)

            @pl.when((g % 2 == 1) & (g < n_g))
            def _():
                work(g, tinB, semB)

            return carry

        lax.fori_loop(0, last_cnt // G, g_body, 0)

        if rem:
            @pl.when(wid == NW - 1)
            def _():
                pltpu.sync_copy(tail_hbm, tail_v)
                pltpu.sync_copy(
                    tail_v, out_hbm.at[pl.ds(n_full * 128 * E, rem * E)]
                )

    return k


def _emb_kernel(N, E, L, NC, NS, CH):
    NW = NC * NS
    rows_per_w = N // NW
    n_ch = rows_per_w // CH
    reps = CH // L
    mesh = plsc.VectorSubcoreMesh(core_axis_name="c", subcore_axis_name="s")

    @functools.partial(
        pl.kernel,
        mesh=mesh,
        compiler_params=pltpu.CompilerParams(use_tc_tiling_on_sc=False),
        out_type=jax.ShapeDtypeStruct((N, 128), jnp.float32),
        scratch_types=[
            pltpu.VMEM((CH,), jnp.int32),
            pltpu.VMEM((CH, E), jnp.float32),
            pltpu.VMEM((L, E), jnp.float32),
            pltpu.SemaphoreType.DMA,
        ],
    )
    def k(x_hbm, tok_hbm, pos_hbm, out_hbm, idx_v, rows_v, pos_v, sem):
        wid = lax.axis_index("s") * NC + lax.axis_index("c")
        base = wid * rows_per_w
        pltpu.sync_copy(pos_hbm, pos_v)

        def chunk_body(c, carry):
            cb = base + c * CH
            pltpu.sync_copy(x_hbm.at[pl.ds(cb, CH)], idx_v)
            pltpu.async_copy(tok_hbm.at[idx_v], rows_v, sem).wait()

            def pos_body(p, carry2):
                for h in range(E // 16):
                    cs = pl.ds(h * 16, 16)
                    pv = pos_v[p, cs]
                    for j in range(reps):
                        r = j * L + p
                        rows_v[r, cs] = rows_v[r, cs] + pv
                return carry2

            lax.fori_loop(0, L, pos_body, 0, unroll=2)
            pltpu.sync_copy(rows_v, out_hbm.at[pl.ds(cb, CH), pl.ds(0, E)])
            return carry

        lax.fori_loop(0, n_ch, chunk_body, 0)

    return k


def kernel(x, token_table, pos_table):
    B, L = x.shape
    V, E = token_table.shape
    N = B * L
    x_flat = x.reshape(N).astype(jnp.int32)
    detile = _detile_kernel(V, E, NC=2, NS=16)
    n_full = V // 128
    tail = token_table[n_full * 128:].reshape(-1)
    tok_flat = detile(token_table.T, tail)
    k = _emb_kernel(N, E, L, NC=2, NS=16, CH=8 * L)
    out = k(x_flat, tok_flat.reshape(V, E), pos_table)
    return out[:, :E].reshape(B, L, E)


# speedup vs baseline: 1.4132x; 1.4128x over previous
import functools

import jax
import jax.numpy as jnp
from jax import lax
from jax.experimental import pallas as pl
from jax.experimental.pallas import tpu as pltpu
from jax.experimental.pallas import tpu_sc as plsc


def _emb_kernel(N, E, L, NC, NS, CH):
    NW = NC * NS
    rows_per_w = N // NW
    n_ch = rows_per_w // CH
    reps = CH // L
    mesh = plsc.VectorSubcoreMesh(core_axis_name="c", subcore_axis_name="s")

    @functools.partial(
        pl.kernel,
        mesh=mesh,
        compiler_params=pltpu.CompilerParams(use_tc_tiling_on_sc=False),
        out_type=jax.ShapeDtypeStruct((N, 128), jnp.float32),
        scratch_types=[
            pltpu.VMEM((CH,), jnp.int32),
            pltpu.VMEM((CH, E), jnp.float32),
            pltpu.VMEM((L, E), jnp.float32),
            pltpu.SemaphoreType.DMA,
        ],
    )
    def k(x_hbm, tok_hbm, pos_hbm, out_hbm, idx_v, rows_v, pos_v, sem):
        wid = lax.axis_index("s") * NC + lax.axis_index("c")
        base = wid * rows_per_w
        pltpu.sync_copy(pos_hbm, pos_v)

        def chunk_body(c, carry):
            cb = base + c * CH
            pltpu.sync_copy(x_hbm.at[pl.ds(cb, CH)], idx_v)
            pltpu.async_copy(tok_hbm.at[idx_v], rows_v, sem).wait()

            def pos_body(p, carry2):
                for h in range(E // 16):
                    cs = pl.ds(h * 16, 16)
                    pv = pos_v[p, cs]
                    for j in range(reps):
                        r = j * L + p
                        rows_v[r, cs] = rows_v[r, cs] + pv
                return carry2

            lax.fori_loop(0, L, pos_body, 0, unroll=2)
            pltpu.sync_copy(rows_v, out_hbm.at[pl.ds(cb, CH), pl.ds(0, E)])
            return carry

        lax.fori_loop(0, n_ch, chunk_body, 0)

    return k


def kernel(x, token_table, pos_table):
    B, L = x.shape
    V, E = token_table.shape
    N = B * L
    x_flat = x.reshape(N).astype(jnp.int32)
    k = _emb_kernel(N, E, L, NC=2, NS=16, CH=8 * L)
    out = k(x_flat, token_table, pos_table)
    return out[:, :E].reshape(B, L, E)


# speedup vs baseline: 1.5477x; 1.0952x over previous
import functools

import jax
import jax.numpy as jnp
from jax import lax
from jax.experimental import pallas as pl
from jax.experimental.pallas import tpu as pltpu
from jax.experimental.pallas import tpu_sc as plsc


def _detile_tc(V, E, Wb=2048):
    Q = Wb // 512
    grid = (V + Wb - 1) // Wb
    rows = grid * (Wb * E // 128)

    def body(tin_ref, out_ref):
        xT = tin_ref[...].T
        out_ref[...] = jnp.concatenate(
            [xT[512 * q:512 * (q + 1)] for q in range(Q)], axis=1
        )

    return rows, pl.pallas_call(
        body,
        grid=(grid,),
        in_specs=[pl.BlockSpec((E, Wb), lambda i: (0, i))],
        out_specs=pl.BlockSpec((Wb * E // 128, 128), lambda i: (i, 0)),
        out_shape=jax.ShapeDtypeStruct((rows, 128), jnp.float32),
    )


def _emb_kernel(N, E, L, NC, NS, CH):
    NW = NC * NS
    rows_per_w = N // NW
    n_ch = rows_per_w // CH
    reps = CH // L
    mesh = plsc.VectorSubcoreMesh(core_axis_name="c", subcore_axis_name="s")

    @functools.partial(
        pl.kernel,
        mesh=mesh,
        compiler_params=pltpu.CompilerParams(use_tc_tiling_on_sc=False),
        out_type=jax.ShapeDtypeStruct((N, 128), jnp.float32),
        scratch_types=[
            pltpu.VMEM((CH,), jnp.int32),
            pltpu.VMEM((CH, E), jnp.float32),
            pltpu.VMEM((L, E), jnp.float32),
            pltpu.SemaphoreType.DMA,
        ],
    )
    def k(x_hbm, tok_hbm, pos_hbm, out_hbm, idx_v, rows_v, pos_v, sem):
        wid = lax.axis_index("s") * NC + lax.axis_index("c")
        base = wid * rows_per_w
        pltpu.sync_copy(pos_hbm, pos_v)

        def chunk_body(c, carry):
            cb = base + c * CH
            pltpu.sync_copy(x_hbm.at[pl.ds(cb, CH)], idx_v)

            def remap_body(kk, carry2):
                for u in range(10):
                    s = pl.ds((kk * 10 + u) * 16, 16)
                    iv = idx_v[s]
                    idx_v[s] = (
                        ((iv >> 11) << 11)
                        + ((iv & 511) << 2)
                        + ((iv & 2047) >> 9)
                    )
                return carry2

            lax.fori_loop(0, CH // 160, remap_body, 0)
            pltpu.async_copy(tok_hbm.at[idx_v], rows_v, sem).wait()

            def pos_body(p, carry2):
                for h in range(E // 16):
                    cs = pl.ds(h * 16, 16)
                    pv = pos_v[p, cs]
                    for j in range(reps):
                        r = j * L + p
                        rows_v[r, cs] = rows_v[r, cs] + pv
                return carry2

            lax.fori_loop(0, L, pos_body, 0, unroll=2)
            pltpu.sync_copy(rows_v, out_hbm.at[pl.ds(cb, CH), pl.ds(0, E)])
            return carry

        lax.fori_loop(0, n_ch, chunk_body, 0)

    return k


def kernel(x, token_table, pos_table):
    B, L = x.shape
    V, E = token_table.shape
    N = B * L
    x_flat = x.reshape(N).astype(jnp.int32)
    rows, detile = _detile_tc(V, E)
    tok_perm = detile(token_table.T).reshape(rows * (128 // E), E)
    k = _emb_kernel(N, E, L, NC=2, NS=16, CH=8 * L)
    out = k(x_flat, tok_perm, pos_table)
    return out[:, :E].reshape(B, L, E)


# speedup vs baseline: 2.0073x; 1.2970x over previous
import functools

import jax
import jax.numpy as jnp
from jax import lax
from jax.experimental import pallas as pl
from jax.experimental.pallas import tpu as pltpu
from jax.experimental.pallas import tpu_sc as plsc


def _detile_tc(V, E, Wb=8192):
    Q = 128 // E
    P = Wb // Q
    grid = (V + Wb - 1) // Wb
    rows = grid * (Wb * E // 128)

    def body(tin_ref, out_ref):
        xT = tin_ref[...].T
        out_ref[...] = jnp.concatenate(
            [xT[P * q:P * (q + 1)] for q in range(Q)], axis=1
        )

    return rows, pl.pallas_call(
        body,
        grid=(grid,),
        in_specs=[pl.BlockSpec((E, Wb), lambda i: (0, i))],
        out_specs=pl.BlockSpec((Wb * E // 128, 128), lambda i: (i, 0)),
        out_shape=jax.ShapeDtypeStruct((rows, 128), jnp.float32),
    )


def _emb_kernel(N, E, L, NC, NS, CH, Wb):
    NW = NC * NS
    rows_per_w = N // NW
    n_ch = rows_per_w // CH
    reps = CH // L
    mesh = plsc.VectorSubcoreMesh(core_axis_name="c", subcore_axis_name="s")

    @functools.partial(
        pl.kernel,
        mesh=mesh,
        compiler_params=pltpu.CompilerParams(use_tc_tiling_on_sc=False),
        out_type=jax.ShapeDtypeStruct((N, 128), jnp.float32),
        scratch_types=[
            pltpu.VMEM((CH,), jnp.int32),
            pltpu.VMEM((CH, E), jnp.float32),
            pltpu.VMEM((L, E), jnp.float32),
            pltpu.SemaphoreType.DMA,
        ],
    )
    def k(x_hbm, tok_hbm, pos_hbm, out_hbm, idx_v, rows_v, pos_v, sem):
        wid = lax.axis_index("s") * NC + lax.axis_index("c")
        base = wid * rows_per_w
        pltpu.sync_copy(pos_hbm, pos_v)

        def chunk_body(c, carry):
            cb = base + c * CH
            pltpu.sync_copy(x_hbm.at[pl.ds(cb, CH)], idx_v)

            lwb = Wb.bit_length() - 1

            def remap_body(kk, carry2):
                for u in range(10):
                    s = pl.ds((kk * 10 + u) * 16, 16)
                    iv = idx_v[s]
                    idx_v[s] = (
                        ((iv >> lwb) << lwb)
                        + ((iv & (Wb // 4 - 1)) << 2)
                        + ((iv & (Wb - 1)) >> (lwb - 2))
                    )
                return carry2

            lax.fori_loop(0, CH // 160, remap_body, 0)
            pltpu.async_copy(tok_hbm.at[idx_v], rows_v, sem).wait()

            def pos_body(p, carry2):
                for h in range(E // 16):
                    cs = pl.ds(h * 16, 16)
                    pv = pos_v[p, cs]
                    for j in range(reps):
                        r = j * L + p
                        rows_v[r, cs] = rows_v[r, cs] + pv
                return carry2

            lax.fori_loop(0, L, pos_body, 0, unroll=2)
            pltpu.sync_copy(rows_v, out_hbm.at[pl.ds(cb, CH), pl.ds(0, E)])
            return carry

        lax.fori_loop(0, n_ch, chunk_body, 0)

    return k


def kernel(x, token_table, pos_table):
    B, L = x.shape
    V, E = token_table.shape
    N = B * L
    x_flat = x.reshape(N).astype(jnp.int32)
    Wb = 8192
    rows, detile = _detile_tc(V, E, Wb)
    tok_perm = detile(token_table.T).reshape(rows * (128 // E), E)
    k = _emb_kernel(N, E, L, NC=2, NS=16, CH=8 * L, Wb=Wb)
    out = k(x_flat, tok_perm, pos_table)
    return out[:, :E].reshape(B, L, E)


# speedup vs baseline: 2.0208x; 1.0067x over previous
import functools

import jax
import jax.numpy as jnp
from jax import lax
from jax.experimental import pallas as pl
from jax.experimental.pallas import tpu as pltpu
from jax.experimental.pallas import tpu_sc as plsc


def _detile_tc(V, E, Wb=8192):
    Q = 128 // E
    P = Wb // Q
    grid = (V + Wb - 1) // Wb
    rows = grid * (Wb * E // 128)

    def body(tin_ref, out_ref):
        xT = tin_ref[...].T
        out_ref[...] = jnp.concatenate(
            [xT[P * q:P * (q + 1)] for q in range(Q)], axis=1
        )

    return rows, pl.pallas_call(
        body,
        grid=(grid,),
        in_specs=[pl.BlockSpec((E, Wb), lambda i: (0, i))],
        out_specs=pl.BlockSpec((Wb * E // 128, 128), lambda i: (i, 0)),
        out_shape=jax.ShapeDtypeStruct((rows, 128), jnp.float32),
    )


def _emb_kernel(N, E, L, NC, NS, CH, Wb):
    NW = NC * NS
    rows_per_w = N // NW
    n_ch = rows_per_w // CH
    reps = CH // L
    mesh = plsc.VectorSubcoreMesh(core_axis_name="c", subcore_axis_name="s")

    @functools.partial(
        pl.kernel,
        mesh=mesh,
        compiler_params=pltpu.CompilerParams(use_tc_tiling_on_sc=False),
        out_type=jax.ShapeDtypeStruct((N, 128), jnp.float32),
        scratch_types=[
            pltpu.VMEM((CH,), jnp.int32),
            pltpu.VMEM((CH, E), jnp.float32),
            pltpu.VMEM((L, E), jnp.float32),
            pltpu.SemaphoreType.DMA,
        ],
    )
    def k(x_hbm, tok_hbm, pos_hbm, out_hbm, idx_v, rows_v, pos_v, sem):
        wid = lax.axis_index("s") * NC + lax.axis_index("c")
        base = wid * rows_per_w
        pltpu.sync_copy(pos_hbm, pos_v)

        def chunk_body(c, carry):
            cb = base + c * CH
            pltpu.sync_copy(x_hbm.at[pl.ds(cb, CH)], idx_v)

            lwb = Wb.bit_length() - 1

            def remap_body(kk, carry2):
                for u in range(10):
                    s = pl.ds((kk * 10 + u) * 16, 16)
                    iv = idx_v[s]
                    idx_v[s] = (
                        ((iv >> lwb) << lwb)
                        + ((iv & (Wb // 4 - 1)) << 2)
                        + ((iv & (Wb - 1)) >> (lwb - 2))
                    )
                return carry2

            lax.fori_loop(0, CH // 160, remap_body, 0)
            pltpu.async_copy(tok_hbm.at[idx_v], rows_v, sem).wait()

            def pos_body(p, carry2):
                for h in range(E // 16):
                    cs = pl.ds(h * 16, 16)
                    pv = pos_v[p, cs]
                    for j in range(reps):
                        r = j * L + p
                        rows_v[r, cs] = rows_v[r, cs] + pv
                return carry2

            lax.fori_loop(0, L, pos_body, 0, unroll=2)
            pltpu.sync_copy(rows_v, out_hbm.at[pl.ds(cb, CH), pl.ds(0, E)])
            return carry

        lax.fori_loop(0, n_ch, chunk_body, 0)

    return k


def kernel(x, token_table, pos_table):
    B, L = x.shape
    V, E = token_table.shape
    N = B * L
    x_flat = x.reshape(N).astype(jnp.int32)
    Wb = 16384
    rows, detile = _detile_tc(V, E, Wb)
    tok_perm = detile(token_table.T).reshape(rows * (128 // E), E)
    k = _emb_kernel(N, E, L, NC=2, NS=16, CH=8 * L, Wb=Wb)
    out = k(x_flat, tok_perm, pos_table)
    return out[:, :E].reshape(B, L, E)
